# baseline (speedup 1.0000x reference)
import functools

import jax
import jax.numpy as jnp
from jax import lax
from jax.experimental import pallas as pl
from jax.experimental.pallas import tpu as pltpu
from jax.experimental.pallas import tpu_sc as plsc

B = 16384
D = 64
NC = 2
NS = 16
NW = NC * NS
BPW = B // NW
CHUNK = 128
NCHUNK = BPW // CHUNK


def _body(u_idx_hbm, i_idx_hbm, user_emb_hbm, ub_hbm, item_emb_hbm, ib_hbm,
          out_hbm,
          u_idx_v, i_idx_v, u_rows, i_rows, ub_v, ib_v, p_v, out_v, sem):
    wid = lax.axis_index("s") * NC + lax.axis_index("c")
    base = wid * BPW

    for j in range(NCHUNK):
        pltpu.sync_copy(u_idx_hbm.at[pl.ds(base + j * CHUNK, CHUNK)], u_idx_v.at[j])
        pltpu.sync_copy(i_idx_hbm.at[pl.ds(base + j * CHUNK, CHUNK)], i_idx_v.at[j])

    copies = []
    for j in range(NCHUNK):
        sl = pl.ds(j * CHUNK, CHUNK)
        copies.append(pltpu.async_copy(user_emb_hbm.at[u_idx_v.at[j]], u_rows.at[sl], sem))
        copies.append(pltpu.async_copy(item_emb_hbm.at[i_idx_v.at[j]], i_rows.at[sl], sem))
        copies.append(pltpu.async_copy(ub_hbm.at[u_idx_v.at[j]], ub_v.at[sl], sem))
        copies.append(pltpu.async_copy(ib_hbm.at[i_idx_v.at[j]], ib_v.at[sl], sem))
    for c in copies:
        c.wait()

    def row(r, _):
        p = u_rows[r, pl.ds(0, 16)] * i_rows[r, pl.ds(0, 16)]
        for c0 in range(16, D, 16):
            p = p + u_rows[r, pl.ds(c0, 16)] * i_rows[r, pl.ds(c0, 16)]
        p_v[pl.ds(r * 16, 16)] = p
        return 0

    lax.fori_loop(0, BPW, row, 0)

    lanes = lax.iota(jnp.int32, 16)

    def group(g, _):
        sl = pl.ds(g * 16, 16)
        base_idx = (g * 16 + lanes) * 16
        x = plsc.load_gather(p_v, [base_idx])
        for c in range(1, 16):
            x = x + plsc.load_gather(p_v, [base_idx + c])
        x = x + ub_v[sl] + ib_v[sl]
        out_v[sl] = 1.0 / (1.0 + jnp.exp(-x))
        return 0

    lax.fori_loop(0, BPW // 16, group, 0)

    pltpu.sync_copy(out_v, out_hbm.at[pl.ds(base, BPW)])


@functools.partial(jax.jit, static_argnames=())
def _run(u_idx, i_idx, user_emb, ub, item_emb, ib):
    mesh = plsc.VectorSubcoreMesh(core_axis_name="c", subcore_axis_name="s",
                                  num_cores=NC, num_subcores=NS)
    f = pl.kernel(
        _body,
        out_type=jax.ShapeDtypeStruct((B,), jnp.float32),
        mesh=mesh,
        compiler_params=pltpu.CompilerParams(needs_layout_passes=False,
                                             use_tc_tiling_on_sc=False),
        scratch_types=[
            pltpu.VMEM((NCHUNK, CHUNK), jnp.int32),
            pltpu.VMEM((NCHUNK, CHUNK), jnp.int32),
            pltpu.VMEM((BPW, D), jnp.float32),
            pltpu.VMEM((BPW, D), jnp.float32),
            pltpu.VMEM((BPW,), jnp.float32),
            pltpu.VMEM((BPW,), jnp.float32),
            pltpu.VMEM((BPW * 16,), jnp.float32),
            pltpu.VMEM((BPW,), jnp.float32),
            pltpu.SemaphoreType.DMA,
        ],
    )
    return f(u_idx, i_idx, user_emb, ub, item_emb, ib)


def kernel(inputs, user_embedding, user_bias, item_embedding, item_bias):
    u_idx = inputs[:, 0]
    i_idx = inputs[:, 1]
    out = _run(u_idx, i_idx, user_embedding, user_bias[:, 0],
               item_embedding, item_bias[:, 0])
    return out[:, None]
